# Initial kernel scaffold; baseline (speedup 1.0000x reference)
import jax
import jax.numpy as jnp
from jax.experimental import pallas as pl


def kernel(y, info, A, gamma, theta, a_par, v, vu, theta_init):
    raise NotImplementedError("write your pallas kernel here")


# speedup vs baseline: 16.8263x; 16.8263x over previous
import functools

import jax
import jax.numpy as jnp
from jax.experimental import pallas as pl
from jax.experimental.pallas import tpu as pltpu

_K = 4
_TOPK = 512
_EPS = 0.01
_ROWS = 128


def _kth_largest(zabs):
    rows = zabs.shape[0]
    bits = jax.lax.bitcast_convert_type(zabs, jnp.int32)
    lo = jnp.zeros((rows, 1), jnp.int32)
    hi = jnp.full((rows, 1), 0x7F800001, jnp.int32)

    def step(_, carry):
        lo, hi = carry
        mid = lo + ((hi - lo) >> 1)
        cnt = jnp.sum((bits >= mid).astype(jnp.int32), axis=1, keepdims=True)
        pred = cnt >= _TOPK
        return jnp.where(pred, mid, lo), jnp.where(pred, hi, mid)

    lo, hi = jax.lax.fori_loop(0, 31, step, (lo, hi))
    return jax.lax.bitcast_convert_type(lo, jnp.float32)


def _soft_threshold(z, theta):
    zabs = jnp.abs(z)
    kth = _kth_largest(zabs)
    soft = jnp.sign(z) * jax.nn.relu(zabs - theta)
    return jnp.where(zabs > kth, z, soft)


def _body(y_ref, A_ref, gamma_ref, theta_ref, a_par_ref, v_ref, vu_ref,
          out_ref):
    y = y_ref[...]
    A = A_ref[...]

    yA = jax.lax.dot_general(y, A, (((1,), (0,)), ((), ())),
                             preferred_element_type=jnp.float32)
    z = gamma_ref[0] * yA
    x_ = _soft_threshold(z, theta_ref[0])
    over = 1.0 + a_par_ref[0] / (jnp.abs(x_) + _EPS)
    x = over * x_

    for i in range(1, _K):
        t = theta_ref[i]
        gain = 1.0 + t * vu_ref[i] * jnp.exp(-v_ref[i] * jnp.abs(x))
        g = gain * x
        a = jax.lax.dot_general(g, A, (((1,), (1,)), ((), ())),
                                preferred_element_type=jnp.float32)
        b = a - y
        c = jax.lax.dot_general(b, A, (((1,), (0,)), ((), ())),
                                preferred_element_type=jnp.float32)
        z = x - gamma_ref[i] * c
        x_ = _soft_threshold(z, t)
        over = 1.0 + a_par_ref[i] / (jnp.abs(x_ - x) + _EPS)
        x = over * x_ + (1.0 - over) * x

    out_ref[...] = x


@jax.jit
def kernel(y, info, A, gamma, theta, a_par, v, vu, theta_init):
    batch, m = y.shape
    n = A.shape[1]
    smem = pl.BlockSpec(memory_space=pltpu.SMEM)
    x = pl.pallas_call(
        _body,
        grid=(batch // _ROWS,),
        in_specs=[
            pl.BlockSpec((_ROWS, m), lambda i: (i, 0)),
            pl.BlockSpec((m, n), lambda i: (0, 0)),
            smem, smem, smem, smem, smem,
        ],
        out_specs=pl.BlockSpec((_ROWS, n), lambda i: (i, 0)),
        out_shape=jax.ShapeDtypeStruct((batch, n), jnp.float32),
        compiler_params=pltpu.CompilerParams(
            dimension_semantics=("parallel",),
            vmem_limit_bytes=100 * 1024 * 1024,
        ),
    )(y, A, gamma, theta, a_par, v, vu)
    zk = jnp.zeros((_K, 1), jnp.float32)
    return x, zk, zk
